# baseline (speedup 1.0000x reference)
import jax
import jax.numpy as jnp
from jax.experimental import pallas as pl
from jax.experimental.pallas import tpu as pltpu

T = 8192
D = 2048
DFF = 8192
E = 8
R = 16
ER = E * R

BT = 512
BF = 2048
CH = 8
BC = BF // CH


def _fused(x_ref, w1_ref, w2_ref, b1_ref, b2_ref, wr_ref, br_ref,
           a_ref, b_ref, alpha_ref, out_ref, g_ref):
    j = pl.program_id(1)
    x = x_ref[...]
    for k in range(CH):
        sl = slice(k * BC, (k + 1) * BC)
        h = jnp.dot(x, w1_ref[:, sl], preferred_element_type=jnp.float32)
        g_ref[:, sl] = jax.nn.gelu(h + b1_ref[:, sl]).astype(jnp.bfloat16)
    p = jnp.dot(g_ref[...], w2_ref[...],
                preferred_element_type=jnp.float32)

    le = jnp.dot(x, wr_ref[...], preferred_element_type=jnp.float32)
    le = le + br_ref[...]
    ex = jnp.exp(le - jnp.max(le, axis=-1, keepdims=True))
    eidx = jax.lax.broadcasted_iota(jnp.int32, ex.shape, 1) // R
    v1 = jnp.max(ex, axis=-1, keepdims=True)
    i1 = jnp.min(jnp.where(ex == v1, eidx, E), axis=-1, keepdims=True)
    m1 = eidx == i1
    ex2 = jnp.where(m1, -1.0, ex)
    v2 = jnp.max(ex2, axis=-1, keepdims=True)
    i2 = jnp.min(jnp.where(ex2 == v2, eidx, E), axis=-1, keepdims=True)
    m2 = eidx == i2
    combine = (jnp.where(m1, v1, 0.0) + jnp.where(m2, v2, 0.0)) / (v1 + v2)
    ha = jax.nn.gelu(jnp.dot(x, a_ref[...],
                             preferred_element_type=jnp.float32))
    hg = (ha * combine).astype(jnp.bfloat16)
    moe = jnp.dot(hg, b_ref[...], preferred_element_type=jnp.float32)

    first = j == 0
    extra = b2_ref[...] + alpha_ref[0, 0] * moe
    out_ref[...] = p + jnp.where(first, extra, out_ref[...])


def kernel(x, W1, b1, W2, b2, Wr, br, A, B, alpha):
    xb = x.astype(jnp.bfloat16)
    w1b = W1.astype(jnp.bfloat16)
    w2b = W2.astype(jnp.bfloat16)
    wr_exp = jnp.repeat(Wr, R, axis=1).astype(jnp.bfloat16)
    br_exp = jnp.repeat(br, R).reshape(1, ER)
    a2d = A.transpose(1, 0, 2).reshape(D, ER).astype(jnp.bfloat16)
    b2d = B.reshape(ER, D).astype(jnp.bfloat16)
    b1r = b1.reshape(1, DFF)
    b2r = b2.reshape(1, D)
    alpha2d = alpha.reshape(1, 1)

    grid = (T // BT, DFF // BF)
    return pl.pallas_call(
        _fused,
        grid=grid,
        in_specs=[
            pl.BlockSpec((BT, D), lambda i, j: (i, 0)),
            pl.BlockSpec((D, BF), lambda i, j: (0, j)),
            pl.BlockSpec((BF, D), lambda i, j: (j, 0)),
            pl.BlockSpec((1, BF), lambda i, j: (0, j)),
            pl.BlockSpec((1, D), lambda i, j: (0, 0)),
            pl.BlockSpec((D, ER), lambda i, j: (0, 0)),
            pl.BlockSpec((1, ER), lambda i, j: (0, 0)),
            pl.BlockSpec((D, ER), lambda i, j: (0, 0)),
            pl.BlockSpec((ER, D), lambda i, j: (0, 0)),
            pl.BlockSpec((1, 1), lambda i, j: (0, 0)),
        ],
        out_specs=pl.BlockSpec((BT, D), lambda i, j: (i, 0)),
        out_shape=jax.ShapeDtypeStruct((T, D), jnp.float32),
        scratch_shapes=[pltpu.VMEM((BT, BF), jnp.bfloat16)],
        compiler_params=pltpu.CompilerParams(
            dimension_semantics=("parallel", "arbitrary"),
        ),
    )(xb, w1b, w2b, b1r, b2r, wr_exp, br_exp, a2d, b2d, alpha2d)


# speedup vs baseline: 1.2092x; 1.2092x over previous
import jax
import jax.numpy as jnp
from jax.experimental import pallas as pl
from jax.experimental.pallas import tpu as pltpu

T = 8192
D = 2048
DFF = 8192
E = 8
R = 16
ER = E * R

BT = 512
BF = 2048
CH = 8
BC = BF // CH


def _fused(x_ref, w1_ref, w2_ref, b1_ref, b2_ref, wr_ref, br_ref,
           a_ref, b_ref, alpha_ref, out_ref, g_ref):
    j = pl.program_id(1)
    x = x_ref[...]
    for k in range(CH):
        sl = slice(k * BC, (k + 1) * BC)
        h = jnp.dot(x, w1_ref[:, sl], preferred_element_type=jnp.float32)
        g_ref[:, sl] = jax.nn.gelu((h + b1_ref[:, sl]).astype(jnp.bfloat16))
    p = jnp.dot(g_ref[...], w2_ref[...],
                preferred_element_type=jnp.float32)

    @pl.when(j == 0)
    def _first():
        le = jnp.dot(x, wr_ref[...], preferred_element_type=jnp.float32)
        le = le + br_ref[...]
        ex = jnp.exp(le - jnp.max(le, axis=-1, keepdims=True))
        eidx = jax.lax.broadcasted_iota(jnp.int32, ex.shape, 1) // R
        v1 = jnp.max(ex, axis=-1, keepdims=True)
        i1 = jnp.min(jnp.where(ex == v1, eidx, E), axis=-1, keepdims=True)
        m1 = eidx == i1
        ex2 = jnp.where(m1, -1.0, ex)
        v2 = jnp.max(ex2, axis=-1, keepdims=True)
        i2 = jnp.min(jnp.where(ex2 == v2, eidx, E), axis=-1, keepdims=True)
        m2 = eidx == i2
        combine = (jnp.where(m1, v1, 0.0) + jnp.where(m2, v2, 0.0)) / (v1 + v2)
        ha = jax.nn.gelu(jnp.dot(x, a_ref[...],
                                 preferred_element_type=jnp.float32))
        hg = (ha * combine).astype(jnp.bfloat16)
        moe = jnp.dot(hg, b_ref[...], preferred_element_type=jnp.float32)
        out_ref[...] = p + b2_ref[...] + alpha_ref[0, 0] * moe

    @pl.when(j != 0)
    def _rest():
        out_ref[...] += p


def kernel(x, W1, b1, W2, b2, Wr, br, A, B, alpha):
    xb = x.astype(jnp.bfloat16)
    w1b = W1.astype(jnp.bfloat16)
    w2b = W2.astype(jnp.bfloat16)
    wr_exp = jnp.repeat(Wr, R, axis=1).astype(jnp.bfloat16)
    br_exp = jnp.repeat(br, R).reshape(1, ER)
    a2d = A.transpose(1, 0, 2).reshape(D, ER).astype(jnp.bfloat16)
    b2d = B.reshape(ER, D).astype(jnp.bfloat16)
    b1r = b1.reshape(1, DFF)
    b2r = b2.reshape(1, D)
    alpha2d = alpha.reshape(1, 1)

    grid = (T // BT, DFF // BF)
    return pl.pallas_call(
        _fused,
        grid=grid,
        in_specs=[
            pl.BlockSpec((BT, D), lambda i, j: (i, 0)),
            pl.BlockSpec((D, BF), lambda i, j: (0, j)),
            pl.BlockSpec((BF, D), lambda i, j: (j, 0)),
            pl.BlockSpec((1, BF), lambda i, j: (0, j)),
            pl.BlockSpec((1, D), lambda i, j: (0, 0)),
            pl.BlockSpec((D, ER), lambda i, j: (0, 0)),
            pl.BlockSpec((1, ER), lambda i, j: (0, 0)),
            pl.BlockSpec((D, ER), lambda i, j: (0, 0)),
            pl.BlockSpec((ER, D), lambda i, j: (0, 0)),
            pl.BlockSpec((1, 1), lambda i, j: (0, 0)),
        ],
        out_specs=pl.BlockSpec((BT, D), lambda i, j: (i, 0)),
        out_shape=jax.ShapeDtypeStruct((T, D), jnp.float32),
        scratch_shapes=[pltpu.VMEM((BT, BF), jnp.bfloat16)],
        compiler_params=pltpu.CompilerParams(
            dimension_semantics=("parallel", "arbitrary"),
        ),
    )(xb, w1b, w2b, b1r, b2r, wr_exp, br_exp, a2d, b2d, alpha2d)


# speedup vs baseline: 1.2094x; 1.0001x over previous
import jax
import jax.numpy as jnp
from jax.experimental import pallas as pl
from jax.experimental.pallas import tpu as pltpu

T = 8192
D = 2048
DFF = 8192
E = 8
R = 16
ER = E * R

BT = 512
BF = 2048
CH = 4
BC = BF // CH


def _fused(x_ref, w1_ref, w2_ref, b1_ref, b2_ref, wr_ref, br_ref,
           a_ref, b_ref, alpha_ref, out_ref, g_ref):
    j = pl.program_id(1)
    x = x_ref[...]
    for k in range(CH):
        sl = slice(k * BC, (k + 1) * BC)
        h = jnp.dot(x, w1_ref[:, sl], preferred_element_type=jnp.float32)
        g_ref[:, sl] = jax.nn.gelu((h + b1_ref[:, sl]).astype(jnp.bfloat16))
    p = jnp.dot(g_ref[...], w2_ref[...],
                preferred_element_type=jnp.float32)

    @pl.when(j == 0)
    def _first():
        le = jnp.dot(x, wr_ref[...], preferred_element_type=jnp.float32)
        le = le + br_ref[...]
        ex = jnp.exp(le - jnp.max(le, axis=-1, keepdims=True))
        eidx = jax.lax.broadcasted_iota(jnp.int32, ex.shape, 1) // R
        v1 = jnp.max(ex, axis=-1, keepdims=True)
        i1 = jnp.min(jnp.where(ex == v1, eidx, E), axis=-1, keepdims=True)
        m1 = eidx == i1
        ex2 = jnp.where(m1, -1.0, ex)
        v2 = jnp.max(ex2, axis=-1, keepdims=True)
        i2 = jnp.min(jnp.where(ex2 == v2, eidx, E), axis=-1, keepdims=True)
        m2 = eidx == i2
        combine = (jnp.where(m1, v1, 0.0) + jnp.where(m2, v2, 0.0)) / (v1 + v2)
        ha = jax.nn.gelu(jnp.dot(x, a_ref[...],
                                 preferred_element_type=jnp.float32))
        hg = (ha * combine).astype(jnp.bfloat16)
        moe = jnp.dot(hg, b_ref[...], preferred_element_type=jnp.float32)
        out_ref[...] = p + b2_ref[...] + alpha_ref[0, 0] * moe

    @pl.when(j != 0)
    def _rest():
        out_ref[...] += p


def kernel(x, W1, b1, W2, b2, Wr, br, A, B, alpha):
    xb = x.astype(jnp.bfloat16)
    w1b = W1.astype(jnp.bfloat16)
    w2b = W2.astype(jnp.bfloat16)
    wr_exp = jnp.repeat(Wr, R, axis=1).astype(jnp.bfloat16)
    br_exp = jnp.repeat(br, R).reshape(1, ER)
    a2d = A.transpose(1, 0, 2).reshape(D, ER).astype(jnp.bfloat16)
    b2d = B.reshape(ER, D).astype(jnp.bfloat16)
    b1r = b1.reshape(1, DFF)
    b2r = b2.reshape(1, D)
    alpha2d = alpha.reshape(1, 1)

    grid = (T // BT, DFF // BF)
    return pl.pallas_call(
        _fused,
        grid=grid,
        in_specs=[
            pl.BlockSpec((BT, D), lambda i, j: (i, 0)),
            pl.BlockSpec((D, BF), lambda i, j: (0, j)),
            pl.BlockSpec((BF, D), lambda i, j: (j, 0)),
            pl.BlockSpec((1, BF), lambda i, j: (0, j)),
            pl.BlockSpec((1, D), lambda i, j: (0, 0)),
            pl.BlockSpec((D, ER), lambda i, j: (0, 0)),
            pl.BlockSpec((1, ER), lambda i, j: (0, 0)),
            pl.BlockSpec((D, ER), lambda i, j: (0, 0)),
            pl.BlockSpec((ER, D), lambda i, j: (0, 0)),
            pl.BlockSpec((1, 1), lambda i, j: (0, 0)),
        ],
        out_specs=pl.BlockSpec((BT, D), lambda i, j: (i, 0)),
        out_shape=jax.ShapeDtypeStruct((T, D), jnp.float32),
        scratch_shapes=[pltpu.VMEM((BT, BF), jnp.bfloat16)],
        compiler_params=pltpu.CompilerParams(
            dimension_semantics=("parallel", "arbitrary"),
        ),
    )(xb, w1b, w2b, b1r, b2r, wr_exp, br_exp, a2d, b2d, alpha2d)
